# baseline (speedup 1.0000x reference)
import jax
import jax.numpy as jnp
from jax import lax
from jax.experimental import pallas as pl
from jax.experimental.pallas import tpu as pltpu, tpu_sc as plsc

B = 4096
T = 200
D = 64
NC = 2
NS = 16
NW = NC * NS

CB = 128
NUNITS = T * (B // CB)
U_PER_W = NUNITS // NW
NBUF = 4
L = 16


def _sc_body(x_hbm, perm_hbm, signs_hbm, out_hbm,
             perm_v, signs_v,
             in_v0, in_v1, in_v2, in_v3, out_v0, out_v1, out_v2, out_v3,
             in_sem0, in_sem1, in_sem2, in_sem3,
             out_sem0, out_sem1, out_sem2, out_sem3):
    in_bufs = [in_v0, in_v1, in_v2, in_v3]
    out_bufs = [out_v0, out_v1, out_v2, out_v3]
    in_sems = [in_sem0, in_sem1, in_sem2, in_sem3]
    out_sems = [out_sem0, out_sem1, out_sem2, out_sem3]
    wid = lax.axis_index("s") * NC + lax.axis_index("c")
    pltpu.sync_copy(perm_hbm, perm_v)
    pltpu.sync_copy(signs_hbm, signs_v)
    base = wid * U_PER_W
    ncb = B // CB
    cols = [jax.lax.iota(jnp.int32, L) + L * j for j in range(CB // L)]

    def hbm_slice(ref, u):
        t = u // ncb
        c = (u % ncb) * CB
        return ref.at[t, :, pl.ds(c, CB)]

    for b in range(NBUF):
        pltpu.async_copy(hbm_slice(x_hbm, base + b), in_bufs[b], in_sems[b])

    def compute(b):
        def row_body(d, _):
            dsplat = jnp.full((L,), d, jnp.int32)
            rsplat = plsc.load_gather(perm_v, [dsplat])
            ssplat = plsc.load_gather(signs_v, [dsplat])
            for j in range(CB // L):
                v = plsc.load_gather(in_bufs[b], [rsplat, cols[j]])
                out_bufs[b][d, pl.ds(L * j, L)] = v * ssplat
            return 0

        plsc.parallel_loop(0, D, 1, unroll=8, carry=jnp.int32(0))(row_body)

    def unit_pair(i2, _):
        for b in range(NBUF):
            u = base + i2 * NBUF + b
            pltpu.make_async_copy(
                hbm_slice(x_hbm, u), in_bufs[b], in_sems[b]).wait()
            @pl.when(i2 > 0)
            def _():
                pltpu.make_async_copy(
                    out_bufs[b], hbm_slice(out_hbm, u - NBUF),
                    out_sems[b]).wait()
            compute(b)
            pltpu.async_copy(out_bufs[b], hbm_slice(out_hbm, u), out_sems[b])
            @pl.when(i2 * NBUF + b + NBUF < U_PER_W)
            def _():
                pltpu.async_copy(
                    hbm_slice(x_hbm, u + NBUF), in_bufs[b], in_sems[b])
        return 0

    lax.fori_loop(0, U_PER_W // NBUF, unit_pair, 0)
    for b in range(NBUF):
        pltpu.make_async_copy(
            out_bufs[b], hbm_slice(out_hbm, base + U_PER_W - NBUF + b),
            out_sems[b]).wait()


@jax.jit
def kernel(x, perm, signs):
    mesh = plsc.VectorSubcoreMesh(
        core_axis_name="c", subcore_axis_name="s", num_cores=NC, num_subcores=NS
    )
    run = pl.kernel(
        _sc_body,
        out_type=jax.ShapeDtypeStruct((T, D, B), jnp.float32),
        mesh=mesh,
        scratch_types=[
            pltpu.VMEM((D,), jnp.int32),
            pltpu.VMEM((D,), jnp.float32),
            pltpu.VMEM((D, CB), jnp.float32),
            pltpu.VMEM((D, CB), jnp.float32),
            pltpu.VMEM((D, CB), jnp.float32),
            pltpu.VMEM((D, CB), jnp.float32),
            pltpu.VMEM((D, CB), jnp.float32),
            pltpu.VMEM((D, CB), jnp.float32),
            pltpu.VMEM((D, CB), jnp.float32),
            pltpu.VMEM((D, CB), jnp.float32),
            pltpu.SemaphoreType.DMA,
            pltpu.SemaphoreType.DMA,
            pltpu.SemaphoreType.DMA,
            pltpu.SemaphoreType.DMA,
            pltpu.SemaphoreType.DMA,
            pltpu.SemaphoreType.DMA,
            pltpu.SemaphoreType.DMA,
            pltpu.SemaphoreType.DMA,
        ],
        compiler_params=pltpu.CompilerParams(
            needs_layout_passes=False,
            use_tc_tiling_on_sc=True,
        ),
    )
    yt = run(jnp.transpose(x, (1, 2, 0)), perm, signs)
    return jnp.transpose(yt, (2, 0, 1))
